# Initial kernel scaffold; baseline (speedup 1.0000x reference)
import jax
import jax.numpy as jnp
from jax import lax
from jax.experimental import pallas as pl
from jax.experimental.pallas import tpu as pltpu
from jax.experimental.pallas import tpu_sc as plsc

B = 16384
D = 128
L = 16
NC = 2
NS = 16
NW = NC * NS
BPW = B // NW
CHUNK = 128
NCHUNK = BPW // CHUNK
DV = D // L


def _gmf_body(uidx_hbm, iidx_hbm, uw_hbm, iw_hbm, w_hbm,
              out_hbm, ss_hbm,
              uidx0, uidx1, iidx0, iidx1,
              ub0, ub1, ib0, ib1,
              wv, outb, ssb, sem0, sem1):
    wid = lax.axis_index("s") * NC + lax.axis_index("c")
    base = wid * BPW

    uidxb = (uidx0, uidx1)
    iidxb = (iidx0, iidx1)
    ub = (ub0, ub1)
    ib = (ib0, ib1)
    sems = (sem0, sem1)

    pltpu.sync_copy(w_hbm, wv)
    wc = [wv[pl.ds(L * k, L)] for k in range(DV)]

    def start(c, slot):
        off = base + c * CHUNK
        pltpu.sync_copy(uidx_hbm.at[pl.ds(off, CHUNK)], uidxb[slot])
        pltpu.sync_copy(iidx_hbm.at[pl.ds(off, CHUNK)], iidxb[slot])
        cu = pltpu.async_copy(uw_hbm.at[uidxb[slot]], ub[slot], sems[slot])
        ci = pltpu.async_copy(iw_hbm.at[iidxb[slot]], ib[slot], sems[slot])
        return cu, ci

    def compute(c, slot, ss):
        ubuf = ub[slot]
        ibuf = ib[slot]

        def group_body(g, ss):
            r0 = g * L
            outvec = jnp.zeros((L,), jnp.float32)
            lane = lax.iota(jnp.int32, L)
            for l in range(L):
                r = r0 + l
                acc = None
                for k in range(DV):
                    u = ubuf[r, pl.ds(L * k, L)]
                    v = ibuf[r, pl.ds(L * k, L)]
                    t = u * v
                    tw = t * wc[k]
                    acc = tw if acc is None else acc + tw
                    ss = ss + u * u + v * v
                rowsum = jnp.sum(acc)
                outvec = jnp.where(lane == l, rowsum, outvec)
            outb[pl.ds(c * CHUNK + r0, L)] = outvec
            return ss

        return lax.fori_loop(0, CHUNK // L, group_body, ss)

    ss = jnp.zeros((L,), jnp.float32)
    pend = start(0, 0)
    for c in range(NCHUNK):
        nxt = start(c + 1, (c + 1) % 2) if c + 1 < NCHUNK else None
        pend[0].wait()
        pend[1].wait()
        ss = compute(c, c % 2, ss)
        pend = nxt

    ssb[...] = ss
    pltpu.sync_copy(outb, out_hbm.at[pl.ds(base, BPW)])
    pltpu.sync_copy(ssb, ss_hbm.at[wid])


@jax.jit
def _gmf(uidx, iidx, uid_w, iid_w, w):
    mesh = plsc.VectorSubcoreMesh(core_axis_name="c", subcore_axis_name="s")
    fn = pl.kernel(
        _gmf_body,
        out_type=(
            jax.ShapeDtypeStruct((B,), jnp.float32),
            jax.ShapeDtypeStruct((NW, L), jnp.float32),
        ),
        mesh=mesh,
        scratch_types=[
            pltpu.VMEM((CHUNK,), jnp.int32),
            pltpu.VMEM((CHUNK,), jnp.int32),
            pltpu.VMEM((CHUNK,), jnp.int32),
            pltpu.VMEM((CHUNK,), jnp.int32),
            pltpu.VMEM((CHUNK, D), jnp.float32),
            pltpu.VMEM((CHUNK, D), jnp.float32),
            pltpu.VMEM((CHUNK, D), jnp.float32),
            pltpu.VMEM((CHUNK, D), jnp.float32),
            pltpu.VMEM((D,), jnp.float32),
            pltpu.VMEM((BPW,), jnp.float32),
            pltpu.VMEM((L,), jnp.float32),
            pltpu.SemaphoreType.DMA,
            pltpu.SemaphoreType.DMA,
        ],
    )
    return fn(uidx, iidx, uid_w, iid_w, w)


def kernel(x, uid_w, iid_w, lin_w):
    xi = x.astype(jnp.int32)
    out, ss = _gmf(xi[:, 0], xi[:, 1], uid_w, iid_w, lin_w.reshape(D))
    emb_loss = jnp.sqrt(jnp.sum(ss)) / jnp.float32(B)
    return (out, emb_loss)


# speedup vs baseline: 1.4238x; 1.4238x over previous
import jax
import jax.numpy as jnp
from jax import lax
from jax.experimental import pallas as pl
from jax.experimental.pallas import tpu as pltpu
from jax.experimental.pallas import tpu_sc as plsc

B = 16384
D = 128
L = 16
NC = 2
NS = 16
NW = NC * NS
BPW = B // NW
CHUNK = 128
NCHUNK = BPW // CHUNK
DV = D // L


def _gmf_body(uidx_hbm, iidx_hbm, uw_hbm, iw_hbm, w_hbm,
              out_hbm, ss_hbm,
              uidx0, uidx1, iidx0, iidx1,
              ub0, ub1, ib0, ib1,
              wv, outb, ssb, tbuf, sem0, sem1):
    wid = lax.axis_index("s") * NC + lax.axis_index("c")
    base = wid * BPW

    uidxb = (uidx0, uidx1)
    iidxb = (iidx0, iidx1)
    ub = (ub0, ub1)
    ib = (ib0, ib1)
    sems = (sem0, sem1)

    pltpu.sync_copy(w_hbm, wv)
    wc = [wv[pl.ds(L * k, L)] for k in range(DV)]

    def start(c, slot):
        off = base + c * CHUNK
        pltpu.sync_copy(uidx_hbm.at[pl.ds(off, CHUNK)], uidxb[slot])
        pltpu.sync_copy(iidx_hbm.at[pl.ds(off, CHUNK)], iidxb[slot])
        cu = pltpu.async_copy(uw_hbm.at[uidxb[slot]], ub[slot], sems[slot])
        ci = pltpu.async_copy(iw_hbm.at[iidxb[slot]], ib[slot], sems[slot])
        return cu, ci

    col0 = lax.iota(jnp.int32, L) * L

    def compute(c, slot, carry):
        ubuf = ub[slot]
        ibuf = ib[slot]

        def group_body(g, carry):
            ssu, ssv = carry
            r0 = g * L
            for l in range(L):
                r = r0 + l
                acc = None
                for k in range(DV):
                    u = ubuf[r, pl.ds(L * k, L)]
                    v = ibuf[r, pl.ds(L * k, L)]
                    t = u * v
                    tw = t * wc[k]
                    acc = tw if acc is None else acc + tw
                    ssu = ssu + u * u
                    ssv = ssv + v * v
                tbuf[pl.ds(l * L, L)] = acc
            s = None
            for j in range(L):
                colv = plsc.load_gather(tbuf, [col0 + j])
                s = colv if s is None else s + colv
            outb[pl.ds(c * CHUNK + r0, L)] = s
            return (ssu, ssv)

        return lax.fori_loop(0, CHUNK // L, group_body, carry)

    zeros = jnp.zeros((L,), jnp.float32)
    carry = (zeros, zeros)
    pend = start(0, 0)
    for c in range(NCHUNK):
        nxt = start(c + 1, (c + 1) % 2) if c + 1 < NCHUNK else None
        pend[0].wait()
        pend[1].wait()
        carry = compute(c, c % 2, carry)
        pend = nxt

    ssb[...] = carry[0] + carry[1]
    pltpu.sync_copy(outb, out_hbm.at[pl.ds(base, BPW)])
    pltpu.sync_copy(ssb, ss_hbm.at[wid])


@jax.jit
def _gmf(uidx, iidx, uid_w, iid_w, w):
    mesh = plsc.VectorSubcoreMesh(core_axis_name="c", subcore_axis_name="s")
    fn = pl.kernel(
        _gmf_body,
        out_type=(
            jax.ShapeDtypeStruct((B,), jnp.float32),
            jax.ShapeDtypeStruct((NW, L), jnp.float32),
        ),
        mesh=mesh,
        compiler_params=pltpu.CompilerParams(needs_layout_passes=False),
        scratch_types=[
            pltpu.VMEM((CHUNK,), jnp.int32),
            pltpu.VMEM((CHUNK,), jnp.int32),
            pltpu.VMEM((CHUNK,), jnp.int32),
            pltpu.VMEM((CHUNK,), jnp.int32),
            pltpu.VMEM((CHUNK, D), jnp.float32),
            pltpu.VMEM((CHUNK, D), jnp.float32),
            pltpu.VMEM((CHUNK, D), jnp.float32),
            pltpu.VMEM((CHUNK, D), jnp.float32),
            pltpu.VMEM((D,), jnp.float32),
            pltpu.VMEM((BPW,), jnp.float32),
            pltpu.VMEM((L,), jnp.float32),
            pltpu.VMEM((L * L,), jnp.float32),
            pltpu.SemaphoreType.DMA,
            pltpu.SemaphoreType.DMA,
        ],
    )
    return fn(uidx, iidx, uid_w, iid_w, w)


def kernel(x, uid_w, iid_w, lin_w):
    xi = x.astype(jnp.int32)
    out, ss = _gmf(xi[:, 0], xi[:, 1], uid_w, iid_w, lin_w.reshape(D))
    emb_loss = jnp.sqrt(jnp.sum(ss)) / jnp.float32(B)
    return (out, emb_loss)


# speedup vs baseline: 1.6126x; 1.1326x over previous
import jax
import jax.numpy as jnp
from jax import lax
from jax.experimental import pallas as pl
from jax.experimental.pallas import tpu as pltpu
from jax.experimental.pallas import tpu_sc as plsc

B = 16384
D = 128
L = 16
NC = 2
NS = 16
NW = NC * NS
BPW = B // NW
CHUNK = 128
NCHUNK = BPW // CHUNK
DV = D // L


def _gmf_body(uidx_hbm, iidx_hbm, uw_hbm, iw_hbm, w_hbm,
              out_hbm, ss_hbm,
              uidxv, iidxv,
              ub0, ub1, ib0, ib1,
              wv, outb, ssb, tbuf, sem0, sem1):
    wid = lax.axis_index("s") * NC + lax.axis_index("c")
    base = wid * BPW

    ub = (ub0, ub1)
    ib = (ib0, ib1)
    sems = (sem0, sem1)

    pltpu.sync_copy(uidx_hbm.at[pl.ds(base, BPW)], uidxv)
    pltpu.sync_copy(iidx_hbm.at[pl.ds(base, BPW)], iidxv)
    pltpu.sync_copy(w_hbm, wv)
    wc = [wv[pl.ds(L * k, L)] for k in range(DV)]

    def start(c, slot):
        off = c * CHUNK
        cu = pltpu.async_copy(
            uw_hbm.at[uidxv.at[pl.ds(off, CHUNK)]], ub[slot], sems[slot])
        ci = pltpu.async_copy(
            iw_hbm.at[iidxv.at[pl.ds(off, CHUNK)]], ib[slot], sems[slot])
        return cu, ci

    col0 = lax.iota(jnp.int32, L) * L

    def compute(c, slot, carry):
        ubuf = ub[slot]
        ibuf = ib[slot]

        RU = 4

        @plsc.parallel_loop(0, CHUNK // RU, carry=carry)
        def carry(rb, carry):
            ssua, ssub, ssva, ssvb = carry
            rr = rb * RU
            for l in range(RU):
                r = rr + l
                acc = None
                for k in range(DV):
                    u = ubuf[r, pl.ds(L * k, L)]
                    v = ibuf[r, pl.ds(L * k, L)]
                    t = u * v
                    tw = t * wc[k]
                    acc = tw if acc is None else acc + tw
                    if l % 2 == 0:
                        ssua = ssua + u * u
                        ssva = ssva + v * v
                    else:
                        ssub = ssub + u * u
                        ssvb = ssvb + v * v
                tbuf[pl.ds(r * L, L)] = acc
            return (ssua, ssub, ssva, ssvb)

        @plsc.parallel_loop(0, CHUNK // L)
        def _(g):
            colg = col0 + g * (L * L)
            s = None
            for j in range(L):
                colv = plsc.load_gather(tbuf, [colg + j])
                s = colv if s is None else s + colv
            outb[pl.ds(c * CHUNK + g * L, L)] = s

        return carry

    zeros = jnp.zeros((L,), jnp.float32)
    carry = (zeros, zeros, zeros, zeros)
    pend = start(0, 0)
    for c in range(NCHUNK):
        nxt = start(c + 1, (c + 1) % 2) if c + 1 < NCHUNK else None
        pend[0].wait()
        pend[1].wait()
        carry = compute(c, c % 2, carry)
        pend = nxt

    ssb[...] = (carry[0] + carry[1]) + (carry[2] + carry[3])
    pltpu.sync_copy(outb, out_hbm.at[pl.ds(base, BPW)])
    pltpu.sync_copy(ssb, ss_hbm.at[wid])


@jax.jit
def _gmf(uidx, iidx, uid_w, iid_w, w):
    mesh = plsc.VectorSubcoreMesh(core_axis_name="c", subcore_axis_name="s")
    fn = pl.kernel(
        _gmf_body,
        out_type=(
            jax.ShapeDtypeStruct((B,), jnp.float32),
            jax.ShapeDtypeStruct((NW, L), jnp.float32),
        ),
        mesh=mesh,
        compiler_params=pltpu.CompilerParams(needs_layout_passes=False),
        scratch_types=[
            pltpu.VMEM((BPW,), jnp.int32),
            pltpu.VMEM((BPW,), jnp.int32),
            pltpu.VMEM((CHUNK, D), jnp.float32),
            pltpu.VMEM((CHUNK, D), jnp.float32),
            pltpu.VMEM((CHUNK, D), jnp.float32),
            pltpu.VMEM((CHUNK, D), jnp.float32),
            pltpu.VMEM((D,), jnp.float32),
            pltpu.VMEM((BPW,), jnp.float32),
            pltpu.VMEM((L,), jnp.float32),
            pltpu.VMEM((CHUNK * L,), jnp.float32),
            pltpu.SemaphoreType.DMA,
            pltpu.SemaphoreType.DMA,
        ],
    )
    return fn(uidx, iidx, uid_w, iid_w, w)


def kernel(x, uid_w, iid_w, lin_w):
    xi = x.astype(jnp.int32)
    out, ss = _gmf(xi[:, 0], xi[:, 1], uid_w, iid_w, lin_w.reshape(D))
    emb_loss = jnp.sqrt(jnp.sum(ss)) / jnp.float32(B)
    return (out, emb_loss)


# speedup vs baseline: 1.8144x; 1.1252x over previous
import jax
import jax.numpy as jnp
from jax import lax
from jax.experimental import pallas as pl
from jax.experimental.pallas import tpu as pltpu
from jax.experimental.pallas import tpu_sc as plsc

B = 16384
D = 128
L = 16
NC = 2
NS = 16
NW = NC * NS
BPW = B // NW
CHUNK = 128
NCHUNK = BPW // CHUNK
DV = D // L


def _gmf_body(uidx_hbm, iidx_hbm, uw_hbm, iw_hbm, w_hbm,
              out_hbm, ss_hbm,
              uidxv, iidxv,
              ub0, ub1, ib0, ib1,
              wv, outb, ssb, tbuf, sem0, sem1):
    wid = lax.axis_index("s") * NC + lax.axis_index("c")
    base = wid * BPW

    ub = (ub0, ub1)
    ib = (ib0, ib1)
    sems = (sem0, sem1)

    cu0 = pltpu.async_copy(uidx_hbm.at[pl.ds(base, BPW)], uidxv, sem0)
    ci0 = pltpu.async_copy(iidx_hbm.at[pl.ds(base, BPW)], iidxv, sem0)
    cw = pltpu.async_copy(w_hbm, wv, sem0)
    cu0.wait()
    ci0.wait()
    cw.wait()
    wc = [wv[pl.ds(L * k, L)] for k in range(DV)]

    def start(c, slot):
        off = c * CHUNK
        cu = pltpu.async_copy(
            uw_hbm.at[uidxv.at[pl.ds(off, CHUNK)]], ub[slot], sems[slot])
        ci = pltpu.async_copy(
            iw_hbm.at[iidxv.at[pl.ds(off, CHUNK)]], ib[slot], sems[slot])
        return cu, ci

    col0 = lax.iota(jnp.int32, L) * L

    def compute(c, slot, carry):
        ubuf = ub[slot]
        ibuf = ib[slot]

        RU = 2

        @plsc.parallel_loop(0, CHUNK // RU, carry=carry)
        def carry(rb, carry):
            ssua, ssub, ssva, ssvb = carry
            rr = rb * RU
            for l in range(RU):
                r = rr + l
                acc = None
                for k in range(DV):
                    u = ubuf[r, pl.ds(L * k, L)]
                    v = ibuf[r, pl.ds(L * k, L)]
                    t = u * v
                    tw = t * wc[k]
                    acc = tw if acc is None else acc + tw
                    if l % 2 == 0:
                        ssua = ssua + u * u
                        ssva = ssva + v * v
                    else:
                        ssub = ssub + u * u
                        ssvb = ssvb + v * v
                tbuf[pl.ds(r * L, L)] = acc
            return (ssua, ssub, ssva, ssvb)

        @plsc.parallel_loop(0, CHUNK // L)
        def _(g):
            colg = col0 + g * (L * L)
            s = None
            for j in range(L):
                colv = plsc.load_gather(tbuf, [colg + j])
                s = colv if s is None else s + colv
            outb[pl.ds(c * CHUNK + g * L, L)] = s

        return carry

    zeros = jnp.zeros((L,), jnp.float32)
    carry = (zeros, zeros, zeros, zeros)
    pend = start(0, 0)
    for c in range(NCHUNK):
        nxt = start(c + 1, (c + 1) % 2) if c + 1 < NCHUNK else None
        pend[0].wait()
        pend[1].wait()
        carry = compute(c, c % 2, carry)
        pend = nxt

    ssb[...] = (carry[0] + carry[1]) + (carry[2] + carry[3])
    pltpu.sync_copy(outb, out_hbm.at[pl.ds(base, BPW)])
    pltpu.sync_copy(ssb, ss_hbm.at[wid])


@jax.jit
def _gmf(uidx, iidx, uid_w, iid_w, w):
    mesh = plsc.VectorSubcoreMesh(core_axis_name="c", subcore_axis_name="s")
    fn = pl.kernel(
        _gmf_body,
        out_type=(
            jax.ShapeDtypeStruct((B,), jnp.float32),
            jax.ShapeDtypeStruct((NW, L), jnp.float32),
        ),
        mesh=mesh,
        compiler_params=pltpu.CompilerParams(needs_layout_passes=False),
        scratch_types=[
            pltpu.VMEM((BPW,), jnp.int32),
            pltpu.VMEM((BPW,), jnp.int32),
            pltpu.VMEM((CHUNK, D), jnp.float32),
            pltpu.VMEM((CHUNK, D), jnp.float32),
            pltpu.VMEM((CHUNK, D), jnp.float32),
            pltpu.VMEM((CHUNK, D), jnp.float32),
            pltpu.VMEM((D,), jnp.float32),
            pltpu.VMEM((BPW,), jnp.float32),
            pltpu.VMEM((L,), jnp.float32),
            pltpu.VMEM((CHUNK * L,), jnp.float32),
            pltpu.SemaphoreType.DMA,
            pltpu.SemaphoreType.DMA,
        ],
    )
    return fn(uidx, iidx, uid_w, iid_w, w)


def kernel(x, uid_w, iid_w, lin_w):
    xi = x.astype(jnp.int32)
    out, ss = _gmf(xi[:, 0], xi[:, 1], uid_w, iid_w, lin_w.reshape(D))
    emb_loss = jnp.sqrt(jnp.sum(ss)) / jnp.float32(B)
    return (out, emb_loss)
